# baseline (device time: 100799 ns/iter reference)
import jax
import jax.numpy as jnp
from jax import lax
from jax.experimental import pallas as pl
from jax.experimental.pallas import tpu as pltpu

N_DEV = 8


def kernel(x, router_W, route_idx, expert_W, shared_W):
    n_tok, d = x.shape
    e_loc, _, h = expert_W.shape
    n_exp = router_W.shape[1]

    ew_bf = expert_W.astype(jnp.bfloat16)

    def body(x_ref, rw_ref, idx_ref, ew_ref, sw_ref, out_ref,
             comm_ref, send_sems, recv_sems):
        me = lax.axis_index("i")
        left = lax.rem(me - 1 + N_DEV, N_DEV)
        right = lax.rem(me + 1, N_DEV)

        barrier_sem = pltpu.get_barrier_semaphore()
        for nbr in (left, right):
            pl.semaphore_signal(
                barrier_sem, inc=1,
                device_id=(nbr,), device_id_type=pl.DeviceIdType.MESH,
            )
        pl.semaphore_wait(barrier_sem, 2)

        xf = x_ref[...]
        scores = jnp.dot(xf, rw_ref[...],
                         preferred_element_type=jnp.float32)
        m = jnp.max(scores, axis=-1, keepdims=True)
        p = jnp.exp(scores - m)
        probs = p / jnp.sum(p, axis=-1, keepdims=True)
        idx = idx_ref[...]
        cols = lax.broadcasted_iota(jnp.int32, (n_tok, n_exp), 1)
        p_sel = jnp.sum(jnp.where(cols == idx, probs, 0.0),
                        axis=-1, keepdims=True)

        xb = xf.astype(jnp.bfloat16)

        out_ref[...] = jnp.dot(xb, sw_ref[...].astype(jnp.bfloat16),
                               preferred_element_type=jnp.float32)

        comm_ref[0, :, :, :] = ew_ref[...]

        for hop in range(N_DEV):
            if hop < N_DEV - 1:
                rdma = pltpu.make_async_remote_copy(
                    src_ref=comm_ref.at[hop],
                    dst_ref=comm_ref.at[hop + 1],
                    send_sem=send_sems.at[hop],
                    recv_sem=recv_sems.at[hop],
                    device_id=(right,),
                    device_id_type=pl.DeviceIdType.MESH,
                )
                rdma.start()

            src_dev = lax.rem(me - hop + N_DEV, N_DEV)
            acc = out_ref[...]
            for j in range(e_loc):
                ge = src_dev * e_loc + j
                gate = jnp.where(idx == ge, p_sel, 0.0)
                xg = (xf * gate).astype(jnp.bfloat16)
                acc = acc + jnp.dot(xg, comm_ref[hop, j],
                                    preferred_element_type=jnp.float32)
            out_ref[...] = acc

            if hop < N_DEV - 1:
                rdma.wait()

    return pl.pallas_call(
        body,
        out_shape=jax.ShapeDtypeStruct((n_tok, h), jnp.float32),
        in_specs=[
            pl.BlockSpec(memory_space=pltpu.VMEM),
            pl.BlockSpec(memory_space=pltpu.VMEM),
            pl.BlockSpec(memory_space=pltpu.VMEM),
            pl.BlockSpec(memory_space=pltpu.VMEM),
            pl.BlockSpec(memory_space=pltpu.VMEM),
        ],
        out_specs=pl.BlockSpec(memory_space=pltpu.VMEM),
        scratch_shapes=[
            pltpu.VMEM((N_DEV, e_loc, d, h), jnp.bfloat16),
            pltpu.SemaphoreType.DMA((N_DEV - 1,)),
            pltpu.SemaphoreType.DMA((N_DEV - 1,)),
        ],
        compiler_params=pltpu.CompilerParams(collective_id=0),
    )(x, router_W, route_idx, ew_bf, shared_W)


# device time: 63639 ns/iter; 1.5839x vs baseline; 1.5839x over previous
import jax
import jax.numpy as jnp
from jax import lax
from jax.experimental import pallas as pl
from jax.experimental.pallas import tpu as pltpu

N_DEV = 8
FW_HOPS = 4
BW_HOPS = 3


def kernel(x, router_W, route_idx, expert_W, shared_W):
    n_tok, d = x.shape
    e_loc, _, h = expert_W.shape
    n_exp = router_W.shape[1]

    ew_bf = expert_W.astype(jnp.bfloat16)

    def body(x_ref, rw_ref, idx_ref, ew_ref, sw_ref, out_ref,
             fw_ref, bw_ref, fw_send, fw_recv, bw_send, bw_recv):
        me = lax.axis_index("i")
        left = lax.rem(me - 1 + N_DEV, N_DEV)
        right = lax.rem(me + 1, N_DEV)

        barrier_sem = pltpu.get_barrier_semaphore()
        for nbr in (left, right):
            pl.semaphore_signal(
                barrier_sem, inc=1,
                device_id=(nbr,), device_id_type=pl.DeviceIdType.MESH,
            )
        pl.semaphore_wait(barrier_sem, 2)

        xf = x_ref[...]
        scores = jnp.dot(xf, rw_ref[...],
                         preferred_element_type=jnp.float32)
        m = jnp.max(scores, axis=-1, keepdims=True)
        p = jnp.exp(scores - m)
        probs = p / jnp.sum(p, axis=-1, keepdims=True)
        idx = idx_ref[...]
        cols = lax.broadcasted_iota(jnp.int32, (n_tok, n_exp), 1)
        p_sel = jnp.sum(jnp.where(cols == idx, probs, 0.0),
                        axis=-1, keepdims=True)

        xb = xf.astype(jnp.bfloat16)

        out_ref[...] = jnp.dot(xb, sw_ref[...].astype(jnp.bfloat16),
                               preferred_element_type=jnp.float32)

        fw_ref[0, :, :, :] = ew_ref[...]
        bw_ref[0, :, :, :] = ew_ref[...]

        def accumulate(acc, block_ref, src_dev):
            for j in range(e_loc):
                ge = src_dev * e_loc + j
                gate = jnp.where(idx == ge, p_sel, 0.0)
                xg = (xf * gate).astype(jnp.bfloat16)
                acc = acc + jnp.dot(xg, block_ref[j],
                                    preferred_element_type=jnp.float32)
            return acc

        for hop in range(FW_HOPS + 1):
            fw = bw = None
            if hop < FW_HOPS:
                fw = pltpu.make_async_remote_copy(
                    src_ref=fw_ref.at[hop],
                    dst_ref=fw_ref.at[hop + 1],
                    send_sem=fw_send.at[hop],
                    recv_sem=fw_recv.at[hop],
                    device_id=(right,),
                    device_id_type=pl.DeviceIdType.MESH,
                )
                fw.start()
            if hop < BW_HOPS:
                bw = pltpu.make_async_remote_copy(
                    src_ref=bw_ref.at[hop],
                    dst_ref=bw_ref.at[hop + 1],
                    send_sem=bw_send.at[hop],
                    recv_sem=bw_recv.at[hop],
                    device_id=(left,),
                    device_id_type=pl.DeviceIdType.MESH,
                )
                bw.start()

            acc = out_ref[...]
            acc = accumulate(acc, fw_ref.at[hop],
                             lax.rem(me - hop + N_DEV, N_DEV))
            if 1 <= hop <= BW_HOPS:
                acc = accumulate(acc, bw_ref.at[hop],
                                 lax.rem(me + hop, N_DEV))
            out_ref[...] = acc

            if fw is not None:
                fw.wait()
            if bw is not None:
                bw.wait()

    return pl.pallas_call(
        body,
        out_shape=jax.ShapeDtypeStruct((n_tok, h), jnp.float32),
        in_specs=[
            pl.BlockSpec(memory_space=pltpu.VMEM),
            pl.BlockSpec(memory_space=pltpu.VMEM),
            pl.BlockSpec(memory_space=pltpu.VMEM),
            pl.BlockSpec(memory_space=pltpu.VMEM),
            pl.BlockSpec(memory_space=pltpu.VMEM),
        ],
        out_specs=pl.BlockSpec(memory_space=pltpu.VMEM),
        scratch_shapes=[
            pltpu.VMEM((FW_HOPS + 1, e_loc, d, h), jnp.bfloat16),
            pltpu.VMEM((BW_HOPS + 1, e_loc, d, h), jnp.bfloat16),
            pltpu.SemaphoreType.DMA((FW_HOPS,)),
            pltpu.SemaphoreType.DMA((FW_HOPS,)),
            pltpu.SemaphoreType.DMA((BW_HOPS,)),
            pltpu.SemaphoreType.DMA((BW_HOPS,)),
        ],
        compiler_params=pltpu.CompilerParams(collective_id=0),
    )(x, router_W, route_idx, ew_bf, shared_W)


# device time: 49130 ns/iter; 2.0517x vs baseline; 1.2953x over previous
import jax
import jax.numpy as jnp
from jax import lax
from jax.experimental import pallas as pl
from jax.experimental.pallas import tpu as pltpu

N_DEV = 8
RING_HOPS = 3


def kernel(x, router_W, route_idx, expert_W, shared_W):
    n_tok, d = x.shape
    e_loc, _, h = expert_W.shape
    n_exp = router_W.shape[1]

    ew_bf = expert_W.astype(jnp.bfloat16)

    def body(x_ref, rw_ref, idx_ref, ew_ref, sw_ref, out_ref,
             fw_ref, bw_ref, cross_ref,
             fw_send, fw_recv, bw_send, bw_recv, cross_send, cross_recv):
        me = lax.axis_index("i")

        def pos_of_rank(k):
            k = lax.rem(k + N_DEV, N_DEV)
            return jnp.where(k < 4, k, 11 - k)

        r = pos_of_rank(me)
        right = pos_of_rank(r + 1)
        left = pos_of_rank(r - 1)
        partner_rank = jnp.where(
            (r == 0) | (r == 3), 3 - r,
            jnp.where((r == 4) | (r == 7), 11 - r, 7 - r))
        partner = pos_of_rank(partner_rank)

        barrier_sem = pltpu.get_barrier_semaphore()
        for nbr in (left, right, partner):
            pl.semaphore_signal(
                barrier_sem, inc=1,
                device_id=(nbr,), device_id_type=pl.DeviceIdType.MESH,
            )
        pl.semaphore_wait(barrier_sem, 3)

        xf = x_ref[...]
        scores = jnp.dot(xf, rw_ref[...],
                         preferred_element_type=jnp.float32)
        m = jnp.max(scores, axis=-1, keepdims=True)
        p = jnp.exp(scores - m)
        probs = p / jnp.sum(p, axis=-1, keepdims=True)
        idx = idx_ref[...]
        cols = lax.broadcasted_iota(jnp.int32, (n_tok, n_exp), 1)
        p_sel = jnp.sum(jnp.where(cols == idx, probs, 0.0),
                        axis=-1, keepdims=True)

        xb = xf.astype(jnp.bfloat16)

        out_ref[...] = jnp.dot(xb, sw_ref[...].astype(jnp.bfloat16),
                               preferred_element_type=jnp.float32)

        fw_ref[0, :, :, :] = ew_ref[...]
        bw_ref[0, :, :, :] = ew_ref[...]

        def accumulate(acc, block_ref, src_pos):
            base = src_pos * e_loc
            parts = []
            for j in range(e_loc):
                gate = jnp.where(idx == base + j, p_sel, 0.0)
                parts.append(xf * gate)
            xg = jnp.concatenate(parts, axis=1).astype(jnp.bfloat16)
            w = block_ref[...].reshape(e_loc * d, h)
            return acc + jnp.dot(xg, w, preferred_element_type=jnp.float32)

        def ring_rdma(flow_ref, sends, recvs, hop, dst):
            return pltpu.make_async_remote_copy(
                src_ref=flow_ref.at[hop],
                dst_ref=flow_ref.at[hop + 1],
                send_sem=sends.at[hop],
                recv_sem=recvs.at[hop],
                device_id=(dst,),
                device_id_type=pl.DeviceIdType.MESH,
            )

        def cross_rdma(src):
            return pltpu.make_async_remote_copy(
                src_ref=src,
                dst_ref=cross_ref,
                send_sem=cross_send,
                recv_sem=cross_recv,
                device_id=(partner,),
                device_id_type=pl.DeviceIdType.MESH,
            )

        fw = ring_rdma(fw_ref, fw_send, fw_recv, 0, right)
        bw = ring_rdma(bw_ref, bw_send, bw_recv, 0, left)
        fw.start()
        bw.start()
        out_ref[...] = accumulate(out_ref[...], ew_ref, me)
        fw.wait()
        bw.wait()

        fw = ring_rdma(fw_ref, fw_send, fw_recv, 1, right)
        bw = ring_rdma(bw_ref, bw_send, bw_recv, 1, left)
        fw.start()
        bw.start()
        r_even = lax.rem(r, 2) == 0

        @pl.when(r_even)
        def _():
            c = cross_rdma(fw_ref.at[1])
            c.start()

        @pl.when(jnp.logical_not(r_even))
        def _():
            c = cross_rdma(bw_ref.at[1])
            c.start()

        acc = out_ref[...]
        acc = accumulate(acc, fw_ref.at[1], pos_of_rank(r - 1))
        acc = accumulate(acc, bw_ref.at[1], pos_of_rank(r + 1))
        out_ref[...] = acc
        fw.wait()
        bw.wait()
        cross_rdma(fw_ref.at[1]).wait()

        fw = ring_rdma(fw_ref, fw_send, fw_recv, 2, right)
        bw = ring_rdma(bw_ref, bw_send, bw_recv, 2, left)
        fw.start()
        bw.start()
        acc = out_ref[...]
        acc = accumulate(acc, fw_ref.at[2], pos_of_rank(r - 2))
        acc = accumulate(acc, bw_ref.at[2], pos_of_rank(r + 2))
        acc = accumulate(acc, cross_ref, pos_of_rank(r + 4))
        out_ref[...] = acc
        fw.wait()
        bw.wait()

        acc = out_ref[...]
        acc = accumulate(acc, fw_ref.at[3], pos_of_rank(r - 3))
        acc = accumulate(acc, bw_ref.at[3], pos_of_rank(r + 3))
        out_ref[...] = acc

    return pl.pallas_call(
        body,
        out_shape=jax.ShapeDtypeStruct((n_tok, h), jnp.float32),
        in_specs=[
            pl.BlockSpec(memory_space=pltpu.VMEM),
            pl.BlockSpec(memory_space=pltpu.VMEM),
            pl.BlockSpec(memory_space=pltpu.VMEM),
            pl.BlockSpec(memory_space=pltpu.VMEM),
            pl.BlockSpec(memory_space=pltpu.VMEM),
        ],
        out_specs=pl.BlockSpec(memory_space=pltpu.VMEM),
        scratch_shapes=[
            pltpu.VMEM((RING_HOPS + 1, e_loc, d, h), jnp.bfloat16),
            pltpu.VMEM((RING_HOPS + 1, e_loc, d, h), jnp.bfloat16),
            pltpu.VMEM((e_loc, d, h), jnp.bfloat16),
            pltpu.SemaphoreType.DMA((RING_HOPS,)),
            pltpu.SemaphoreType.DMA((RING_HOPS,)),
            pltpu.SemaphoreType.DMA((RING_HOPS,)),
            pltpu.SemaphoreType.DMA((RING_HOPS,)),
            pltpu.SemaphoreType.DMA,
            pltpu.SemaphoreType.DMA,
        ],
        compiler_params=pltpu.CompilerParams(collective_id=0),
    )(x, router_W, route_idx, ew_bf, shared_W)


# device time: 48482 ns/iter; 2.0791x vs baseline; 1.0134x over previous
import jax
import jax.numpy as jnp
from jax import lax
from jax.experimental import pallas as pl
from jax.experimental.pallas import tpu as pltpu

N_DEV = 8
RING_HOPS = 3


def kernel(x, router_W, route_idx, expert_W, shared_W):
    n_tok, d = x.shape
    e_loc, _, h = expert_W.shape
    n_exp = router_W.shape[1]

    ew_bf = expert_W.astype(jnp.bfloat16)

    def body(x_ref, rw_ref, idx_ref, ew_ref, sw_ref, out_ref,
             fw_ref, bw_ref, cross_ref,
             fw_send, fw_recv, bw_send, bw_recv, cross_send, cross_recv):
        me = lax.axis_index("i")

        def pos_of_rank(k):
            k = lax.rem(k + N_DEV, N_DEV)
            return jnp.where(k < 4, k, 11 - k)

        r = pos_of_rank(me)
        right = pos_of_rank(r + 1)
        left = pos_of_rank(r - 1)
        partner_rank = jnp.where(
            (r == 0) | (r == 3), 3 - r,
            jnp.where((r == 4) | (r == 7), 11 - r, 7 - r))
        partner = pos_of_rank(partner_rank)

        barrier_sem = pltpu.get_barrier_semaphore()
        for nbr in (left, right, partner):
            pl.semaphore_signal(
                barrier_sem, inc=1,
                device_id=(nbr,), device_id_type=pl.DeviceIdType.MESH,
            )
        pl.semaphore_wait(barrier_sem, 3)

        fw = pltpu.make_async_remote_copy(
            src_ref=ew_ref, dst_ref=fw_ref.at[1],
            send_sem=fw_send.at[0], recv_sem=fw_recv.at[0],
            device_id=(right,), device_id_type=pl.DeviceIdType.MESH,
        )
        bw = pltpu.make_async_remote_copy(
            src_ref=ew_ref, dst_ref=bw_ref.at[1],
            send_sem=bw_send.at[0], recv_sem=bw_recv.at[0],
            device_id=(left,), device_id_type=pl.DeviceIdType.MESH,
        )
        fw.start()
        bw.start()

        xf = x_ref[...]
        scores = jnp.dot(xf, rw_ref[...],
                         preferred_element_type=jnp.float32)
        m = jnp.max(scores, axis=-1, keepdims=True)
        p = jnp.exp(scores - m)
        probs = p / jnp.sum(p, axis=-1, keepdims=True)
        idx = idx_ref[...]
        cols = lax.broadcasted_iota(jnp.int32, (n_tok, n_exp), 1)
        p_sel = jnp.sum(jnp.where(cols == idx, probs, 0.0),
                        axis=-1, keepdims=True)

        xb = xf.astype(jnp.bfloat16)

        out_ref[...] = jnp.dot(xb, sw_ref[...].astype(jnp.bfloat16),
                               preferred_element_type=jnp.float32)

        def accumulate(acc, block_ref, src_pos):
            base = src_pos * e_loc
            parts = []
            for j in range(e_loc):
                gate = jnp.where(idx == base + j, p_sel, 0.0)
                parts.append(xf * gate)
            xg = jnp.concatenate(parts, axis=1).astype(jnp.bfloat16)
            w = block_ref[...].reshape(e_loc * d, h)
            return acc + jnp.dot(xg, w, preferred_element_type=jnp.float32)

        def ring_rdma(flow_ref, sends, recvs, hop, dst):
            return pltpu.make_async_remote_copy(
                src_ref=flow_ref.at[hop],
                dst_ref=flow_ref.at[hop + 1],
                send_sem=sends.at[hop],
                recv_sem=recvs.at[hop],
                device_id=(dst,),
                device_id_type=pl.DeviceIdType.MESH,
            )

        def cross_rdma(src):
            return pltpu.make_async_remote_copy(
                src_ref=src,
                dst_ref=cross_ref,
                send_sem=cross_send,
                recv_sem=cross_recv,
                device_id=(partner,),
                device_id_type=pl.DeviceIdType.MESH,
            )

        out_ref[...] = accumulate(out_ref[...], ew_ref, me)
        fw.wait()
        bw.wait()

        fw = ring_rdma(fw_ref, fw_send, fw_recv, 1, right)
        bw = ring_rdma(bw_ref, bw_send, bw_recv, 1, left)
        fw.start()
        bw.start()
        r_even = lax.rem(r, 2) == 0

        @pl.when(r_even)
        def _():
            c = cross_rdma(fw_ref.at[1])
            c.start()

        @pl.when(jnp.logical_not(r_even))
        def _():
            c = cross_rdma(bw_ref.at[1])
            c.start()

        acc = out_ref[...]
        acc = accumulate(acc, fw_ref.at[1], pos_of_rank(r - 1))
        acc = accumulate(acc, bw_ref.at[1], pos_of_rank(r + 1))
        out_ref[...] = acc
        fw.wait()
        bw.wait()

        fw = ring_rdma(fw_ref, fw_send, fw_recv, 2, right)
        bw = ring_rdma(bw_ref, bw_send, bw_recv, 2, left)
        fw.start()
        bw.start()
        cross_rdma(fw_ref.at[1]).wait()
        acc = out_ref[...]
        acc = accumulate(acc, fw_ref.at[2], pos_of_rank(r - 2))
        acc = accumulate(acc, bw_ref.at[2], pos_of_rank(r + 2))
        acc = accumulate(acc, cross_ref, pos_of_rank(r + 4))
        out_ref[...] = acc
        fw.wait()
        bw.wait()

        acc = out_ref[...]
        acc = accumulate(acc, fw_ref.at[3], pos_of_rank(r - 3))
        acc = accumulate(acc, bw_ref.at[3], pos_of_rank(r + 3))
        out_ref[...] = acc

    return pl.pallas_call(
        body,
        out_shape=jax.ShapeDtypeStruct((n_tok, h), jnp.float32),
        in_specs=[
            pl.BlockSpec(memory_space=pltpu.VMEM),
            pl.BlockSpec(memory_space=pltpu.VMEM),
            pl.BlockSpec(memory_space=pltpu.VMEM),
            pl.BlockSpec(memory_space=pltpu.VMEM),
            pl.BlockSpec(memory_space=pltpu.VMEM),
        ],
        out_specs=pl.BlockSpec(memory_space=pltpu.VMEM),
        scratch_shapes=[
            pltpu.VMEM((RING_HOPS + 1, e_loc, d, h), jnp.bfloat16),
            pltpu.VMEM((RING_HOPS + 1, e_loc, d, h), jnp.bfloat16),
            pltpu.VMEM((e_loc, d, h), jnp.bfloat16),
            pltpu.SemaphoreType.DMA((RING_HOPS,)),
            pltpu.SemaphoreType.DMA((RING_HOPS,)),
            pltpu.SemaphoreType.DMA((RING_HOPS,)),
            pltpu.SemaphoreType.DMA((RING_HOPS,)),
            pltpu.SemaphoreType.DMA,
            pltpu.SemaphoreType.DMA,
        ],
        compiler_params=pltpu.CompilerParams(collective_id=0),
    )(x, router_W, route_idx, ew_bf, shared_W)


# device time: 42584 ns/iter; 2.3671x vs baseline; 1.1385x over previous
import jax
import jax.numpy as jnp
from jax import lax
from jax.experimental import pallas as pl
from jax.experimental.pallas import tpu as pltpu

N_DEV = 8


def kernel(x, router_W, route_idx, expert_W, shared_W):
    n_tok, d = x.shape
    e_loc, _, h = expert_W.shape
    n_exp = router_W.shape[1]
    half = e_loc // 2

    ew_bf = expert_W.astype(jnp.bfloat16)

    def body(x_ref, rw_ref, idx_ref, ew_ref, sw_ref, out_ref,
             fw_ref, bw_ref, cr0_ref, cr1_ref, cr2_ref,
             send_sems, recv_sems):
        me = lax.axis_index("i")

        def pos_of_rank(k):
            k = lax.rem(k + 2 * N_DEV, N_DEV)
            return jnp.where(k < 4, k, 11 - k)

        r = pos_of_rank(me)
        right = pos_of_rank(r + 1)
        left = pos_of_rank(r - 1)
        r_even = lax.rem(r, 2) == 0
        partner = pos_of_rank(jnp.where(r_even, r + 3, r - 3))

        barrier_sem = pltpu.get_barrier_semaphore()
        for nbr in (left, right, partner):
            pl.semaphore_signal(
                barrier_sem, inc=1,
                device_id=(nbr,), device_id_type=pl.DeviceIdType.MESH,
            )
        pl.semaphore_wait(barrier_sem, 3)

        FW1, FW2, FW3A, BW1, BW2, BW3A, C0, C1, C2 = range(9)

        def rdma(i, src, dst, dev):
            return pltpu.make_async_remote_copy(
                src_ref=src, dst_ref=dst,
                send_sem=send_sems.at[i], recv_sem=recv_sems.at[i],
                device_id=(dev,), device_id_type=pl.DeviceIdType.MESH,
            )

        a_half = pl.ds(0, half)
        b_half = pl.ds(half, half)

        fw1 = rdma(FW1, ew_ref, fw_ref.at[1], right)
        bw1 = rdma(BW1, ew_ref, bw_ref.at[1], left)
        c0 = rdma(C0, ew_ref.at[b_half], cr0_ref, partner)
        fw1.start()
        bw1.start()
        c0.start()

        xf = x_ref[...]
        scores = jnp.dot(xf, rw_ref[...],
                         preferred_element_type=jnp.float32)
        m = jnp.max(scores, axis=-1, keepdims=True)
        p = jnp.exp(scores - m)
        probs = p / jnp.sum(p, axis=-1, keepdims=True)
        idx = idx_ref[...]
        cols = lax.broadcasted_iota(jnp.int32, (n_tok, n_exp), 1)
        p_sel = jnp.sum(jnp.where(cols == idx, probs, 0.0),
                        axis=-1, keepdims=True)

        xb = xf.astype(jnp.bfloat16)

        def accumulate(acc, block_ref, src_pos, j0, nj):
            base = src_pos * e_loc + j0
            parts = []
            for j in range(nj):
                gate = jnp.where(idx == base + j, p_sel, 0.0)
                parts.append(xf * gate)
            xg = jnp.concatenate(parts, axis=1).astype(jnp.bfloat16)
            w = block_ref[...].reshape(nj * d, h)
            return acc + jnp.dot(xg, w, preferred_element_type=jnp.float32)

        acc = jnp.dot(xb, sw_ref[...].astype(jnp.bfloat16),
                      preferred_element_type=jnp.float32)
        acc = accumulate(acc, ew_ref, me, 0, e_loc)
        out_ref[...] = acc

        fw1.wait()
        bw1.wait()

        fw2 = rdma(FW2, fw_ref.at[1], fw_ref.at[2], right)
        bw2 = rdma(BW2, bw_ref.at[1], bw_ref.at[2], left)
        fw2.start()
        bw2.start()

        @pl.when(r_even)
        def _():
            rdma(C1, fw_ref.at[1], cr1_ref, partner).start()

        @pl.when(jnp.logical_not(r_even))
        def _():
            rdma(C1, bw_ref.at[1], cr1_ref, partner).start()

        c0.wait()
        acc = out_ref[...]
        acc = accumulate(acc, fw_ref.at[1], pos_of_rank(r - 1), 0, e_loc)
        acc = accumulate(acc, bw_ref.at[1], pos_of_rank(r + 1), 0, e_loc)
        acc = accumulate(acc, cr0_ref,
                         pos_of_rank(jnp.where(r_even, r + 3, r - 3)),
                         half, half)
        out_ref[...] = acc
        fw2.wait()
        bw2.wait()

        fw3a = rdma(FW3A, fw_ref.at[2, a_half], fw_ref.at[3, a_half], right)
        bw3a = rdma(BW3A, bw_ref.at[2, a_half], bw_ref.at[3, a_half], left)
        fw3a.start()
        bw3a.start()

        @pl.when(r_even)
        def _():
            rdma(C2, fw_ref.at[2, b_half], cr2_ref, partner).start()

        @pl.when(jnp.logical_not(r_even))
        def _():
            rdma(C2, bw_ref.at[2, b_half], cr2_ref, partner).start()

        rdma(C1, fw_ref.at[1], cr1_ref, partner).wait()
        acc = out_ref[...]
        acc = accumulate(acc, fw_ref.at[2], pos_of_rank(r - 2), 0, e_loc)
        acc = accumulate(acc, bw_ref.at[2], pos_of_rank(r + 2), 0, e_loc)
        acc = accumulate(acc, cr1_ref, pos_of_rank(r + 4), 0, e_loc)
        out_ref[...] = acc

        fw3a.wait()
        bw3a.wait()
        rdma(C2, fw_ref.at[2, b_half], cr2_ref, partner).wait()
        acc = out_ref[...]
        acc = accumulate(acc, fw_ref.at[3, a_half], pos_of_rank(r - 3),
                         0, half)
        acc = accumulate(acc, bw_ref.at[3, a_half], pos_of_rank(r + 3),
                         0, half)
        acc = accumulate(acc, cr2_ref,
                         pos_of_rank(jnp.where(r_even, r - 3, r + 3)),
                         half, half)
        out_ref[...] = acc

    return pl.pallas_call(
        body,
        out_shape=jax.ShapeDtypeStruct((n_tok, h), jnp.float32),
        in_specs=[
            pl.BlockSpec(memory_space=pltpu.VMEM),
            pl.BlockSpec(memory_space=pltpu.VMEM),
            pl.BlockSpec(memory_space=pltpu.VMEM),
            pl.BlockSpec(memory_space=pltpu.VMEM),
            pl.BlockSpec(memory_space=pltpu.VMEM),
        ],
        out_specs=pl.BlockSpec(memory_space=pltpu.VMEM),
        scratch_shapes=[
            pltpu.VMEM((4, e_loc, d, h), jnp.bfloat16),
            pltpu.VMEM((4, e_loc, d, h), jnp.bfloat16),
            pltpu.VMEM((half, d, h), jnp.bfloat16),
            pltpu.VMEM((e_loc, d, h), jnp.bfloat16),
            pltpu.VMEM((half, d, h), jnp.bfloat16),
            pltpu.SemaphoreType.DMA((9,)),
            pltpu.SemaphoreType.DMA((9,)),
        ],
        compiler_params=pltpu.CompilerParams(collective_id=0),
    )(x, router_W, route_idx, ew_bf, shared_W)
